# baseline (speedup 1.0000x reference)
import functools

import jax
import jax.numpy as jnp
from jax import lax
from jax.experimental import pallas as pl
from jax.experimental.pallas import tpu as pltpu
from jax.experimental.pallas import tpu_sc as plsc

_NC, _NS = 2, 16
_NW = _NC * _NS
_D = 8
_MAX_IDX = 128


@functools.lru_cache(maxsize=None)
def _build_gather(b, rows, emb):
    assert b % (2 * _NW) == 0
    bpw = b // _NW
    nsteps = bpw // 2
    chunks = [(o, min(_MAX_IDX, rows - o)) for o in range(0, rows, _MAX_IDX)]
    mesh = plsc.VectorSubcoreMesh(
        core_axis_name="c", subcore_axis_name="s",
        num_cores=_NC, num_subcores=_NS)

    def body(flat_hbm, perm_hbm, out_hbm, idx_v, buf_a, buf_b, sem_a, sem_b):
        wid = lax.axis_index("s") * _NC + lax.axis_index("c")
        b0 = wid * bpw
        pltpu.sync_copy(perm_hbm, idx_v)

        def start(batch, buf, sem):
            src = flat_hbm.at[batch]
            for off, sz in chunks:
                pltpu.async_copy(src.at[idx_v.at[pl.ds(off, sz)]],
                                 buf.at[pl.ds(off, sz)], sem)

        def drain(buf, sem):
            pltpu.make_async_copy(flat_hbm.at[0], buf, sem).wait()

        def flush(buf, batch):
            pltpu.sync_copy(buf, out_hbm.at[pl.ds(batch * rows, rows)])

        start(b0, buf_a, sem_a)

        def step(t, carry):
            ba = b0 + 2 * t
            start(ba + 1, buf_b, sem_b)
            drain(buf_a, sem_a)
            flush(buf_a, ba)

            @pl.when(t < nsteps - 1)
            def _():
                start(ba + 2, buf_a, sem_a)

            drain(buf_b, sem_b)
            flush(buf_b, ba + 1)
            return carry

        lax.fori_loop(0, nsteps, step, 0)

    return pl.kernel(
        body,
        out_type=jax.ShapeDtypeStruct((b * rows, emb), jnp.float32),
        mesh=mesh,
        compiler_params=pltpu.CompilerParams(use_tc_tiling_on_sc=False),
        scratch_types=[
            pltpu.VMEM((rows,), jnp.int32),
            pltpu.VMEM((rows, emb), jnp.float32),
            pltpu.VMEM((rows, emb), jnp.float32),
            pltpu.SemaphoreType.DMA,
            pltpu.SemaphoreType.DMA,
        ],
    )


def kernel(M, permutator):
    b, n_cond, n_col, emb = M.shape
    rows = n_cond * n_col
    flat3 = M.reshape(b, rows, emb)
    perm = permutator.astype(jnp.int32)
    out = _build_gather(b, rows, emb)(flat3, perm)
    return out.reshape(b, rows // _D, _D, emb)


# speedup vs baseline: 10.0058x; 10.0058x over previous
import functools

import jax
import jax.numpy as jnp
from jax import lax
from jax.experimental import pallas as pl
from jax.experimental.pallas import tpu as pltpu
from jax.experimental.pallas import tpu_sc as plsc

_NC, _NS = 2, 16
_NW = _NC * _NS
_D = 8


@functools.lru_cache(maxsize=None)
def _build_permute(rows, emb, b):
    assert rows % (2 * _NW) == 0
    spw = rows // _NW
    nsteps = spw // 2
    swin = ((spw + 6) // 8 + 1) * 8 + 16
    mesh = plsc.VectorSubcoreMesh(
        core_axis_name="c", subcore_axis_name="s",
        num_cores=_NC, num_subcores=_NS)

    def body(src_hbm, perm_hbm, out_hbm, idx_v, buf_a, buf_b, sem_a, sem_b):
        wid = lax.axis_index("s") * _NC + lax.axis_index("c")
        s0 = wid * spw
        base = pl.multiple_of((s0 // 8) * 8, 8)
        off = s0 - base
        pltpu.sync_copy(perm_hbm.at[pl.ds(base, swin)], idx_v)

        def gather(i, buf, sem):
            slab = idx_v[pl.ds(off + i, 16)][0]
            pltpu.async_copy(src_hbm.at[pl.ds(slab, 1)], buf, sem)

        def drain(buf, sem):
            pltpu.make_async_copy(src_hbm.at[pl.ds(0, 1)], buf, sem).wait()

        def flush(buf, i):
            pltpu.sync_copy(buf, out_hbm.at[pl.ds(s0 + i, 1)])

        gather(0, buf_a, sem_a)

        def step(t, carry):
            ia = 2 * t
            gather(ia + 1, buf_b, sem_b)
            drain(buf_a, sem_a)
            flush(buf_a, ia)

            @pl.when(t < nsteps - 1)
            def _():
                gather(ia + 2, buf_a, sem_a)

            drain(buf_b, sem_b)
            flush(buf_b, ia + 1)
            return carry

        lax.fori_loop(0, nsteps, step, 0)

    return pl.kernel(
        body,
        out_type=jax.ShapeDtypeStruct((rows, emb, b), jnp.float32),
        mesh=mesh,
        scratch_types=[
            pltpu.VMEM((swin,), jnp.int32),
            pltpu.VMEM((1, emb, b), jnp.float32),
            pltpu.VMEM((1, emb, b), jnp.float32),
            pltpu.SemaphoreType.DMA,
            pltpu.SemaphoreType.DMA,
        ],
    )


def kernel(M, permutator):
    b, n_cond, n_col, emb = M.shape
    rows = n_cond * n_col
    Mv = jnp.transpose(M.reshape(b, rows, emb), (1, 2, 0))
    perm = jnp.pad(permutator.astype(jnp.int32), (0, 32))
    out_v = _build_permute(rows, emb, b)(Mv, perm)
    return jnp.transpose(out_v.reshape(rows // _D, _D, emb, b), (3, 0, 1, 2))
